# Initial kernel scaffold; baseline (speedup 1.0000x reference)
import jax
import jax.numpy as jnp
from jax.experimental import pallas as pl


def kernel(x, edge_index, edge_attr, W1, b1, W2, b2, W3, b3):
    raise NotImplementedError("write your pallas kernel here")


# speedup vs baseline: 9.3318x; 9.3318x over previous
import functools

import jax
import jax.numpy as jnp
from jax import lax
from jax.experimental import pallas as pl
from jax.experimental.pallas import tpu as pltpu
from jax.experimental.pallas import tpu_sc as plsc

NC, NS, LANES = 2, 16, 16
TILES = NC * NS
CHUNK = 128
DP = 128

N = 10000
E = 320000
ET = E + N
CPT = -(-ET // (TILES * CHUNK))
ETP = TILES * CHUNK * CPT
EPT = CPT * CHUNK
RPT = -(-(-(-N // NS)) // CHUNK) * CHUNK
NP_ = RPT * NS


def _mesh():
    return plsc.VectorSubcoreMesh(
        core_axis_name="c", subcore_axis_name="s", num_cores=NC, num_subcores=NS
    )


def _zero_1d(buf, n):
    z = jnp.zeros((LANES,), jnp.float32)
    for j in range(n // LANES):
        buf[pl.ds(j * LANES, LANES)] = z


def _zero_2d(buf, n, d):
    z = jnp.zeros((LANES,), jnp.float32)

    def body(j, carry):
        for kk in range(d // LANES):
            buf[j, pl.ds(kk * LANES, LANES)] = z
        return carry

    lax.fori_loop(0, n, body, 0)


def _deg_body(col_hbm, w_hbm, deg_hbm, colbuf, wbuf, acc):
    cid = lax.axis_index("c")
    sid = lax.axis_index("s")
    wid = cid * NS + sid
    start = sid * RPT
    _zero_1d(wbuf, CHUNK)
    for t in range(RPT // CHUNK):
        pltpu.sync_copy(wbuf, acc.at[pl.ds(start + t * CHUNK, CHUNK)])
    plsc.subcore_barrier()

    def chunk(j, carry):
        base = wid * EPT + j * CHUNK
        pltpu.sync_copy(col_hbm.at[pl.ds(base, CHUNK)], colbuf)
        pltpu.sync_copy(w_hbm.at[pl.ds(base, CHUNK)], wbuf)
        pltpu.sync_copy(wbuf, acc.at[colbuf], add=True)
        return carry

    lax.fori_loop(0, CPT, chunk, 0)
    plsc.subcore_barrier()
    for t in range(RPT // CHUNK):
        pltpu.sync_copy(acc.at[pl.ds(start + t * CHUNK, CHUNK)], wbuf)
        pltpu.sync_copy(wbuf, deg_hbm.at[pl.ds(cid * NP_ + start + t * CHUNK,
                                               CHUNK)])


def _sc_degree(col, w):
    k = pl.kernel(
        _deg_body,
        out_type=jax.ShapeDtypeStruct((NC * NP_,), jnp.float32),
        mesh=_mesh(),
        scratch_types=[
            pltpu.VMEM((CHUNK,), jnp.int32),
            pltpu.VMEM((CHUNK,), jnp.float32),
            pltpu.VMEM_SHARED((NP_,), jnp.float32),
        ],
    )
    return k(col, w)


def _agg_body(DR, g_hbm, row_hbm, col_hbm, w_hbm, out_hbm,
              rowbuf, colbuf, wbuf, rows, acc, sem):
    cid = lax.axis_index("c")
    sid = lax.axis_index("s")
    wid = cid * NS + sid
    start = sid * RPT
    _zero_2d(rows, CHUNK, DP)
    for t in range(RPT // CHUNK):
        pltpu.sync_copy(rows, acc.at[pl.ds(start + t * CHUNK, CHUNK)])
    plsc.subcore_barrier()

    def chunk(j, carry):
        base = wid * EPT + j * CHUNK
        pltpu.sync_copy(row_hbm.at[pl.ds(base, CHUNK)], rowbuf)
        pltpu.sync_copy(col_hbm.at[pl.ds(base, CHUNK)], colbuf)
        pltpu.sync_copy(w_hbm.at[pl.ds(base, CHUNK)], wbuf)
        pltpu.async_copy(g_hbm.at[rowbuf], rows, sem).wait()
        for g in range(CHUNK // LANES):
            w16 = wbuf[pl.ds(g * LANES, LANES)]
            for i in range(LANES):
                e = g * LANES + i
                s = w16[i]
                for kk in range(DR // LANES):
                    sl = pl.ds(kk * LANES, LANES)
                    rows[e, sl] = rows[e, sl] * s
        pltpu.sync_copy(rows, acc.at[colbuf], add=True)
        return carry

    lax.fori_loop(0, CPT, chunk, 0)
    plsc.subcore_barrier()
    for t in range(RPT // CHUNK):
        pltpu.sync_copy(acc.at[pl.ds(start + t * CHUNK, CHUNK)], rows)
        pltpu.sync_copy(rows, out_hbm.at[pl.ds(cid * NP_ + start + t * CHUNK,
                                               CHUNK)])


def _sc_aggregate(g, row, col, w, DR):
    k = pl.kernel(
        functools.partial(_agg_body, DR),
        out_type=jax.ShapeDtypeStruct((NC * NP_, DP), jnp.float32),
        mesh=_mesh(),
        scratch_types=[
            pltpu.VMEM((CHUNK,), jnp.int32),
            pltpu.VMEM((CHUNK,), jnp.int32),
            pltpu.VMEM((CHUNK,), jnp.float32),
            pltpu.VMEM((CHUNK, DP), jnp.float32),
            pltpu.VMEM_SHARED((NP_, DP), jnp.float32),
            pltpu.SemaphoreType.DMA,
        ],
    )
    return k(g, row, col, w)


def _tc1_body(dpt_ref, x_ref, w1_ref, dinv_ref, g1_ref):
    deg = dpt_ref[:, 0:1] + dpt_ref[:, 1:2]
    dinv = jnp.where(deg > 0, lax.rsqrt(deg), 0.0)
    dinv_ref[...] = dinv
    h = lax.dot_general(
        x_ref[...], w1_ref[...], (((1,), (0,)), ((), ())),
        precision=lax.Precision.HIGHEST, preferred_element_type=jnp.float32)
    g1_ref[:, 0:64] = dinv[:N] * h
    g1_ref[:, 64:DP] = jnp.zeros((N, DP - 64), jnp.float32)


def _tc_mid_body(DI, DO, p_ref, dinv_ref, b_ref, w_ref, g_ref):
    dinv = dinv_ref[...]
    p = p_ref[0, :, 0:DI] + p_ref[1, :, 0:DI]
    h = jnp.maximum(dinv * p + b_ref[...], 0.0)
    g_ref[:, 0:DO] = dinv * lax.dot_general(
        h, w_ref[...], (((1,), (0,)), ((), ())),
        precision=lax.Precision.HIGHEST, preferred_element_type=jnp.float32)
    g_ref[:, DO:DP] = jnp.zeros((NP_, DP - DO), jnp.float32)


def _tc_fin_body(p_ref, dinv_ref, b_ref, node_ref, graph_ref):
    p = p_ref[0, :N, 0:32] + p_ref[1, :N, 0:32]
    h = dinv_ref[:N] * p + b_ref[...]
    node_ref[...] = h
    graph_ref[...] = jnp.mean(h, axis=0, keepdims=True)


def kernel(x, edge_index, edge_attr, W1, b1, W2, b2, W3, b3):
    f32 = jnp.float32
    loop = jnp.arange(N, dtype=edge_index.dtype)
    pad = ETP - ET
    row = jnp.concatenate([edge_index[0], loop, jnp.zeros((pad,), edge_index.dtype)])
    col = jnp.concatenate([edge_index[1], loop, jnp.zeros((pad,), edge_index.dtype)])
    w = jnp.concatenate([edge_attr.reshape(-1), jnp.ones((N,), f32),
                         jnp.zeros((pad,), f32)])

    degp = _sc_degree(col, w)
    dpt = degp.reshape(NC, NP_).T

    dinv, g1 = pl.pallas_call(
        _tc1_body,
        out_shape=[jax.ShapeDtypeStruct((NP_, 1), f32),
                   jax.ShapeDtypeStruct((N, DP), f32)],
    )(dpt, x, W1)

    p1 = _sc_aggregate(g1, row, col, w, 64)

    g2 = pl.pallas_call(
        functools.partial(_tc_mid_body, 64, 64),
        out_shape=jax.ShapeDtypeStruct((NP_, DP), f32),
    )(p1.reshape(NC, NP_, DP), dinv, b1.reshape(1, 64), W2)

    p2 = _sc_aggregate(g2, row, col, w, 64)

    g3 = pl.pallas_call(
        functools.partial(_tc_mid_body, 64, 32),
        out_shape=jax.ShapeDtypeStruct((NP_, DP), f32),
    )(p2.reshape(NC, NP_, DP), dinv, b2.reshape(1, 64), W3)

    p3 = _sc_aggregate(g3, row, col, w, 32)

    node, graph = pl.pallas_call(
        _tc_fin_body,
        out_shape=[jax.ShapeDtypeStruct((N, 32), f32),
                   jax.ShapeDtypeStruct((1, 32), f32)],
    )(p3.reshape(NC, NP_, DP), dinv, b3.reshape(1, 32))

    return (node, graph)


# speedup vs baseline: 18.4586x; 1.9780x over previous
import functools

import jax
import jax.numpy as jnp
from jax import lax
from jax.experimental import pallas as pl
from jax.experimental.pallas import tpu as pltpu
from jax.experimental.pallas import tpu_sc as plsc

NC, NS, LANES = 2, 16, 16
TILES = NC * NS
CHUNK = 128
DP = 128

N = 10000
E = 320000
ET = E + N
CPT = -(-ET // (TILES * CHUNK))
CPT += CPT % 2
ETP = TILES * CHUNK * CPT
ETP2 = ETP + 2 * CHUNK
EPT = CPT * CHUNK
RPT = -(-(-(-N // NS)) // CHUNK) * CHUNK
NP_ = RPT * NS


def _mesh():
    return plsc.VectorSubcoreMesh(
        core_axis_name="c", subcore_axis_name="s", num_cores=NC, num_subcores=NS
    )


def _zero_1d(buf, n):
    z = jnp.zeros((LANES,), jnp.float32)
    for j in range(n // LANES):
        buf[pl.ds(j * LANES, LANES)] = z


def _zero_2d(buf, n, d):
    z = jnp.zeros((LANES,), jnp.float32)

    def body(j, carry):
        for kk in range(d // LANES):
            buf[j, pl.ds(kk * LANES, LANES)] = z
        return carry

    lax.fori_loop(0, n, body, 0)


def _deg_body(col_hbm, w_hbm, deg_hbm, colbuf, wbuf, acc):
    cid = lax.axis_index("c")
    sid = lax.axis_index("s")
    wid = cid * NS + sid
    start = sid * RPT
    _zero_1d(wbuf, CHUNK)
    for t in range(RPT // CHUNK):
        pltpu.sync_copy(wbuf, acc.at[pl.ds(start + t * CHUNK, CHUNK)])
    plsc.subcore_barrier()

    def chunk(j, carry):
        base = wid * EPT + j * CHUNK
        pltpu.sync_copy(col_hbm.at[pl.ds(base, CHUNK)], colbuf)
        pltpu.sync_copy(w_hbm.at[pl.ds(base, CHUNK)], wbuf)
        pltpu.sync_copy(wbuf, acc.at[colbuf], add=True)
        return carry

    lax.fori_loop(0, CPT, chunk, 0)
    plsc.subcore_barrier()
    for t in range(RPT // CHUNK):
        pltpu.sync_copy(acc.at[pl.ds(start + t * CHUNK, CHUNK)], wbuf)
        pltpu.sync_copy(wbuf, deg_hbm.at[pl.ds(cid * NP_ + start + t * CHUNK,
                                               CHUNK)])


def _sc_degree(col, w):
    k = pl.kernel(
        _deg_body,
        out_type=jax.ShapeDtypeStruct((NC * NP_,), jnp.float32),
        mesh=_mesh(),
        scratch_types=[
            pltpu.VMEM((CHUNK,), jnp.int32),
            pltpu.VMEM((CHUNK,), jnp.float32),
            pltpu.VMEM_SHARED((NP_,), jnp.float32),
        ],
    )
    return k(col, w)


def _agg_body(DR, g_hbm, row_hbm, col_hbm, w_hbm, out_hbm,
              rA, cA, wA, rB, cB, wB, rowsA, rowsB, acc,
              srA, scA, swA, srB, scB, swB, gsA, gsB):
    cid = lax.axis_index("c")
    sid = lax.axis_index("s")
    wid = cid * NS + sid
    start = sid * RPT

    def idx_start(ch, rb, cb, wb, sr, sc_, sw):
        base = wid * EPT + ch * CHUNK
        pltpu.async_copy(row_hbm.at[pl.ds(base, CHUNK)], rb, sr)
        pltpu.async_copy(col_hbm.at[pl.ds(base, CHUNK)], cb, sc_)
        pltpu.async_copy(w_hbm.at[pl.ds(base, CHUNK)], wb, sw)

    def idx_wait(ch, rb, cb, wb, sr, sc_, sw):
        base = wid * EPT + ch * CHUNK
        pltpu.make_async_copy(row_hbm.at[pl.ds(base, CHUNK)], rb, sr).wait()
        pltpu.make_async_copy(col_hbm.at[pl.ds(base, CHUNK)], cb, sc_).wait()
        pltpu.make_async_copy(w_hbm.at[pl.ds(base, CHUNK)], wb, sw).wait()

    def scale_scatter(rows, wb, cb):
        for g in range(CHUNK // LANES):
            w16 = wb[pl.ds(g * LANES, LANES)]
            for i in range(LANES):
                e = g * LANES + i
                s = w16[i]
                for kk in range(DR // LANES):
                    sl = pl.ds(kk * LANES, LANES)
                    rows[e, sl] = rows[e, sl] * s
        pltpu.sync_copy(rows, acc.at[cb], add=True)

    idx_start(0, rA, cA, wA, srA, scA, swA)
    idx_start(1, rB, cB, wB, srB, scB, swB)
    _zero_2d(rowsA, CHUNK, DP)
    for t in range(RPT // CHUNK):
        pltpu.sync_copy(rowsA, acc.at[pl.ds(start + t * CHUNK, CHUNK)])
    plsc.subcore_barrier()
    idx_wait(0, rA, cA, wA, srA, scA, swA)
    pltpu.async_copy(g_hbm.at[rA], rowsA, gsA)

    def body(k, carry):
        a = 2 * k
        idx_wait(a + 1, rB, cB, wB, srB, scB, swB)
        pltpu.async_copy(g_hbm.at[rB], rowsB, gsB)
        pltpu.make_async_copy(g_hbm.at[rA], rowsA, gsA).wait()
        scale_scatter(rowsA, wA, cA)
        idx_start(a + 2, rA, cA, wA, srA, scA, swA)
        pltpu.make_async_copy(g_hbm.at[rB], rowsB, gsB).wait()
        scale_scatter(rowsB, wB, cB)
        idx_wait(a + 2, rA, cA, wA, srA, scA, swA)
        pltpu.async_copy(g_hbm.at[rA], rowsA, gsA)
        idx_start(a + 3, rB, cB, wB, srB, scB, swB)
        return carry

    lax.fori_loop(0, CPT // 2, body, 0)
    pltpu.make_async_copy(g_hbm.at[rA], rowsA, gsA).wait()
    idx_wait(CPT + 1, rB, cB, wB, srB, scB, swB)
    plsc.subcore_barrier()
    for t in range(RPT // CHUNK):
        pltpu.sync_copy(acc.at[pl.ds(start + t * CHUNK, CHUNK)], rowsA)
        pltpu.sync_copy(rowsA, out_hbm.at[pl.ds(cid * NP_ + start + t * CHUNK,
                                                CHUNK)])


def _sc_aggregate(g, row, col, w, DR):
    k = pl.kernel(
        functools.partial(_agg_body, DR),
        out_type=jax.ShapeDtypeStruct((NC * NP_, DP), jnp.float32),
        mesh=_mesh(),
        scratch_types=[
            pltpu.VMEM((CHUNK,), jnp.int32),
            pltpu.VMEM((CHUNK,), jnp.int32),
            pltpu.VMEM((CHUNK,), jnp.float32),
            pltpu.VMEM((CHUNK,), jnp.int32),
            pltpu.VMEM((CHUNK,), jnp.int32),
            pltpu.VMEM((CHUNK,), jnp.float32),
            pltpu.VMEM((CHUNK, DP), jnp.float32),
            pltpu.VMEM((CHUNK, DP), jnp.float32),
            pltpu.VMEM_SHARED((NP_, DP), jnp.float32),
            pltpu.SemaphoreType.DMA,
            pltpu.SemaphoreType.DMA,
            pltpu.SemaphoreType.DMA,
            pltpu.SemaphoreType.DMA,
            pltpu.SemaphoreType.DMA,
            pltpu.SemaphoreType.DMA,
            pltpu.SemaphoreType.DMA,
            pltpu.SemaphoreType.DMA,
        ],
    )
    return k(g, row, col, w)


def _tc1_body(dpt_ref, x_ref, w1_ref, dinv_ref, g1_ref):
    deg = dpt_ref[:, 0:1] + dpt_ref[:, 1:2]
    dinv = jnp.where(deg > 0, lax.rsqrt(deg), 0.0)
    dinv_ref[...] = dinv
    h = lax.dot_general(
        x_ref[...], w1_ref[...], (((1,), (0,)), ((), ())),
        precision=lax.Precision.HIGHEST, preferred_element_type=jnp.float32)
    g1_ref[:, 0:64] = dinv[:N] * h
    g1_ref[:, 64:DP] = jnp.zeros((N, DP - 64), jnp.float32)


def _tc_mid_body(DI, DO, p_ref, dinv_ref, b_ref, w_ref, g_ref):
    dinv = dinv_ref[...]
    p = p_ref[0, :, 0:DI] + p_ref[1, :, 0:DI]
    h = jnp.maximum(dinv * p + b_ref[...], 0.0)
    g_ref[:, 0:DO] = dinv * lax.dot_general(
        h, w_ref[...], (((1,), (0,)), ((), ())),
        precision=lax.Precision.HIGHEST, preferred_element_type=jnp.float32)
    g_ref[:, DO:DP] = jnp.zeros((NP_, DP - DO), jnp.float32)


def _tc_fin_body(p_ref, dinv_ref, b_ref, node_ref, graph_ref):
    p = p_ref[0, :N, 0:32] + p_ref[1, :N, 0:32]
    h = dinv_ref[:N] * p + b_ref[...]
    node_ref[...] = h
    graph_ref[...] = jnp.mean(h, axis=0, keepdims=True)


def kernel(x, edge_index, edge_attr, W1, b1, W2, b2, W3, b3):
    f32 = jnp.float32
    loop = jnp.arange(N, dtype=edge_index.dtype)
    pad = ETP2 - ET
    pad_idx = (jnp.arange(pad) % N).astype(edge_index.dtype)
    row = jnp.concatenate([edge_index[0], loop, pad_idx])
    col = jnp.concatenate([edge_index[1], loop, pad_idx])
    w = jnp.concatenate([edge_attr.reshape(-1), jnp.ones((N,), f32),
                         jnp.zeros((pad,), f32)])

    degp = _sc_degree(col, w)
    dpt = degp.reshape(NC, NP_).T

    dinv, g1 = pl.pallas_call(
        _tc1_body,
        out_shape=[jax.ShapeDtypeStruct((NP_, 1), f32),
                   jax.ShapeDtypeStruct((N, DP), f32)],
    )(dpt, x, W1)

    p1 = _sc_aggregate(g1, row, col, w, 64)

    g2 = pl.pallas_call(
        functools.partial(_tc_mid_body, 64, 64),
        out_shape=jax.ShapeDtypeStruct((NP_, DP), f32),
    )(p1.reshape(NC, NP_, DP), dinv, b1.reshape(1, 64), W2)

    p2 = _sc_aggregate(g2, row, col, w, 64)

    g3 = pl.pallas_call(
        functools.partial(_tc_mid_body, 64, 32),
        out_shape=jax.ShapeDtypeStruct((NP_, DP), f32),
    )(p2.reshape(NC, NP_, DP), dinv, b2.reshape(1, 64), W3)

    p3 = _sc_aggregate(g3, row, col, w, 32)

    node, graph = pl.pallas_call(
        _tc_fin_body,
        out_shape=[jax.ShapeDtypeStruct((N, 32), f32),
                   jax.ShapeDtypeStruct((1, 32), f32)],
    )(p3.reshape(NC, NP_, DP), dinv, b3.reshape(1, 32))

    return (node, graph)
